# baseline (speedup 1.0000x reference)
import functools
import math

import jax
import jax.numpy as jnp
from jax import lax
from jax.experimental import pallas as pl
from jax.experimental.pallas import tpu as pltpu
from jax.experimental.pallas import tpu_sc as plsc

N_NODES = 10000
D = 256
H = 4
HD = D // H
HALF = D // 2

_BN = 1000

_C = 32
_NTILE = 16
_E_TOT = 160000 + N_NODES
_EPT = 10752
_EP = _EPT * _NTILE
_NCHUNK = _EPT // _C

_NL = 5000
_SINK = _NL
_RDL = 320
_SROWS = 80


def _qkv_body(x_ref, wq_ref, wk_ref, wv_ref, q_ref, k_ref, v_ref):
    x = x_ref[...]
    for w_ref, o_ref in ((wq_ref, q_ref), (wk_ref, k_ref), (wv_ref, v_ref)):
        full = jax.lax.dot_general(
            x, w_ref[...], (((1,), (1,)), ((), ())),
            preferred_element_type=jnp.float32)
        o_ref[0, ...] = full[:, :HALF]
        o_ref[1, ...] = full[:, HALF:]


def _qkv_proj(x, Wq, Wk, Wv):
    n = x.shape[0]
    grid = (n // _BN,)
    row_spec = pl.BlockSpec((_BN, D), lambda i: (i, 0))
    w_spec = pl.BlockSpec((D, D), lambda i: (0, 0))
    half_spec = pl.BlockSpec((2, _BN, HALF), lambda i: (0, i, 0))
    return pl.pallas_call(
        _qkv_body,
        grid=grid,
        in_specs=[row_spec, w_spec, w_spec, w_spec],
        out_specs=[half_spec] * 3,
        out_shape=[jax.ShapeDtypeStruct((2, n, HALF), jnp.float32)] * 3,
    )(x, Wq, Wk, Wv)


_sc_mesh = plsc.VectorSubcoreMesh(core_axis_name="c", subcore_axis_name="s")


@functools.partial(
    pl.kernel,
    mesh=_sc_mesh,
    out_type=[
        jax.ShapeDtypeStruct((2 * N_NODES, HALF), jnp.float32),
        jax.ShapeDtypeStruct((4 * _SROWS, HALF), jnp.float32),
    ],
    scratch_types=[
        pltpu.VMEM((2, _C), jnp.int32),
        pltpu.VMEM((2, _C), jnp.int32),
        pltpu.VMEM((2, _C), jnp.int32),
        pltpu.VMEM((2, _C), jnp.int32),
        pltpu.VMEM((2, _C), jnp.int32),
        pltpu.VMEM((2 * _C, HALF), jnp.float32),
        pltpu.VMEM((2 * _C, HALF), jnp.float32),
        pltpu.VMEM((2 * _C, HALF), jnp.float32),
        pltpu.VMEM((_C, HALF), jnp.float32),
        pltpu.VMEM((_C, HALF), jnp.float32),
        pltpu.VMEM_SHARED((_NL + 8, HALF), jnp.float32),
        pltpu.VMEM_SHARED((_SROWS, HALF), jnp.float32),
        pltpu.SemaphoreType.DMA,
        pltpu.SemaphoreType.DMA,
        pltpu.SemaphoreType.DMA,
    ],
)
def _edge_kernel(rowg_hbm, colg_hbm, cols_hbm, qh_hbm, kh_hbm, vh_hbm,
                 zeros_hbm, acc_out, sums_out,
                 rowg_v, colg_v, cols_v, colsS_v, srow_v, qbuf, kbuf, vbuf,
                 mbuf, msum, acc_sh, sums_sh, gsem0, gsem1, ssem):
    c = lax.axis_index("c")
    s = lax.axis_index("s")
    gsems = (gsem0, gsem1)

    lanes = lax.iota(jnp.int32, 16)
    base_idx = c * _EP + s * _EPT
    base_raw0 = s * _EPT

    def stage_and_fire(ch, b):
        base = base_idx + ch * _C
        base_raw = base_raw0 + ch * _C
        pltpu.sync_copy(rowg_hbm.at[pl.ds(base, _C)], rowg_v.at[b])
        pltpu.sync_copy(colg_hbm.at[pl.ds(base, _C)], colg_v.at[b])
        pltpu.sync_copy(cols_hbm.at[pl.ds(base_raw, _C)], cols_v.at[b])
        pltpu.async_copy(qh_hbm.at[rowg_v.at[b]],
                         qbuf.at[pl.ds(b * _C, _C)], gsems[b])
        pltpu.async_copy(kh_hbm.at[colg_v.at[b]],
                         kbuf.at[pl.ds(b * _C, _C)], gsems[b])
        pltpu.async_copy(vh_hbm.at[rowg_v.at[b]],
                         vbuf.at[pl.ds(b * _C, _C)], gsems[b])

    def wait_gathers(b):
        pltpu.make_async_copy(qh_hbm.at[rowg_v.at[b]],
                              qbuf.at[pl.ds(b * _C, _C)], gsems[b]).wait()
        pltpu.make_async_copy(kh_hbm.at[colg_v.at[b]],
                              kbuf.at[pl.ds(b * _C, _C)], gsems[b]).wait()
        pltpu.make_async_copy(vh_hbm.at[rowg_v.at[b]],
                              vbuf.at[pl.ds(b * _C, _C)], gsems[b]).wait()

    def wait_scatters(b):
        pltpu.make_async_copy(mbuf, acc_sh.at[colsS_v.at[b]], ssem).wait()
        pltpu.make_async_copy(msum, sums_sh.at[srow_v.at[b]], ssem).wait()

    def compute(t, b):
        roff = b * _C

        zero = jnp.zeros((16,), jnp.float32)

        def group_body(g, carry2):
            e0 = g * 16
            colraw = cols_v[b, pl.ds(e0, 16)]
            loc = colraw - t * _NL
            col16 = jnp.where((loc >= 0) & (loc < _NL), loc, _SINK)
            srow_v[b, pl.ds(e0, 16)] = col16 >> 6
            colsS_v[b, pl.ds(e0, 16)] = col16
            inrow16 = (col16 & 63) * 2

            for e in range(16):
                eg = e0 + e
                er = roff + eg
                acc0 = zero
                acc1 = zero
                for j in range(4):
                    acc0 = acc0 + (qbuf[er, pl.ds(j * 16, 16)]
                                   * kbuf[er, pl.ds(j * 16, 16)])
                    acc1 = acc1 + (qbuf[er, pl.ds(64 + j * 16, 16)]
                                   * kbuf[er, pl.ds(64 + j * 16, 16)])
                for kk in (1, 2, 4, 8):
                    perm = lanes ^ kk
                    acc0 = acc0 + acc0.at[perm].get(mode="promise_in_bounds")
                    acc1 = acc1 + acc1.at[perm].get(mode="promise_in_bounds")
                p0all = jnp.exp(acc0 * 0.125)
                p1all = jnp.exp(acc1 * 0.125)
                ev = jnp.zeros((16,), jnp.int32) + e
                inrow = inrow16.at[ev].get(mode="promise_in_bounds")
                inlane = inrow & 15
                jbv = inrow >> 4
                ohbase = (jnp.where(lanes == inlane, p0all, 0.0)
                          + jnp.where(lanes == inlane + 1, p1all, 0.0))
                for j in range(8):
                    pe = p0all if j < 4 else p1all
                    mbuf[eg, pl.ds(j * 16, 16)] = (
                        vbuf[er, pl.ds(j * 16, 16)] * pe)
                    dz = jnp.minimum(jnp.abs(jbv - j), 1)
                    msum[eg, pl.ds(j * 16, 16)] = ohbase * (
                        1.0 - dz.astype(jnp.float32))
            return carry2

        for g in range(_C // 16):
            group_body(g, 0)

    def phase_body(t, carry):
        @pl.when(s < _NTILE - 1)
        def _():
            pltpu.sync_copy(zeros_hbm.at[pl.ds(0, _RDL)],
                            acc_sh.at[pl.ds(s * _RDL, _RDL)])

        @pl.when(s == _NTILE - 1)
        def _():
            pltpu.sync_copy(zeros_hbm.at[pl.ds(0, 200)],
                            acc_sh.at[pl.ds(s * _RDL, 200)])

        @pl.when(s == 0)
        def _():
            pltpu.sync_copy(zeros_hbm.at[pl.ds(0, _SROWS)], sums_sh)

        plsc.subcore_barrier()

        stage_and_fire(0, 0)

        def pair_body(i, carry2):
            for b in range(2):
                ch = i * 2 + b
                nb = b ^ 1

                @pl.when(ch + 1 < _NCHUNK)
                def _():
                    stage_and_fire(ch + 1, nb)

                wait_gathers(b)

                @pl.when(ch > 0)
                def _():
                    wait_scatters(nb)

                compute(t, b)
                pltpu.async_copy(mbuf, acc_sh.at[colsS_v.at[b]], ssem,
                                 add=True)
                pltpu.async_copy(msum, sums_sh.at[srow_v.at[b]], ssem,
                                 add=True)
            return carry2

        lax.fori_loop(0, _NCHUNK // 2, pair_body, 0)
        wait_scatters(1)

        plsc.subcore_barrier()

        obase = c * N_NODES + t * _NL

        @pl.when(s < _NTILE - 1)
        def _():
            pltpu.sync_copy(acc_sh.at[pl.ds(s * _RDL, _RDL)],
                            acc_out.at[pl.ds(obase + s * _RDL, _RDL)])

        @pl.when(s == _NTILE - 1)
        def _():
            pltpu.sync_copy(acc_sh.at[pl.ds(s * _RDL, 200)],
                            acc_out.at[pl.ds(obase + s * _RDL, 200)])

        @pl.when(s == 0)
        def _():
            pltpu.sync_copy(
                sums_sh,
                sums_out.at[pl.ds((c * 2 + t) * _SROWS, _SROWS)])

        plsc.subcore_barrier()
        return carry

    lax.fori_loop(0, 2, phase_body, 0)


def _ln(x, g, b):
    mu = jnp.mean(x, axis=-1, keepdims=True)
    var = jnp.mean((x - mu) ** 2, axis=-1, keepdims=True)
    return (x - mu) * jax.lax.rsqrt(var + 1e-5) * g + b


def _erf(z):
    sg = jnp.sign(z)
    z = jnp.abs(z)
    t = 1.0 / (1.0 + 0.3275911 * z)
    poly = t * (0.254829592 + t * (-0.284496736 + t * (1.421413741
               + t * (-1.453152027 + t * 1.061405429))))
    return sg * (1.0 - poly * jnp.exp(-z * z))


def _post_body(out_ref, recip_ref, x_ref, wo_ref, bo_ref, w1_ref, b1_ref,
               w2_ref, b2_ref, g1_ref, be1_ref, g2_ref, be2_ref, y_ref):
    out = out_ref[...] * recip_ref[...]
    attn = jax.lax.dot_general(
        out, wo_ref[...], (((1,), (1,)), ((), ())),
        preferred_element_type=jnp.float32) + bo_ref[...]
    x1 = _ln(x_ref[...] + attn, g1_ref[...], be1_ref[...])
    h = jax.lax.dot_general(
        x1, w1_ref[...], (((1,), (1,)), ((), ())),
        preferred_element_type=jnp.float32) + b1_ref[...]
    h = h * 0.5 * (1.0 + _erf(h * (1.0 / math.sqrt(2.0))))
    ffn = jax.lax.dot_general(
        h, w2_ref[...], (((1,), (1,)), ((), ())),
        preferred_element_type=jnp.float32) + b2_ref[...]
    y_ref[...] = _ln(x1 + ffn, g2_ref[...], be2_ref[...])


def _post(out, recip, x, Wo, bo, W1, b1, W2, b2, g1, be1, g2, be2):
    n = x.shape[0]
    bo, b1, b2, g1, be1, g2, be2 = (a.reshape(1, -1) for a in
                                    (bo, b1, b2, g1, be1, g2, be2))
    grid = (n // _BN,)
    row_spec = pl.BlockSpec((_BN, D), lambda i: (i, 0))

    def full(a):
        return pl.BlockSpec(a.shape, lambda i: (0,) * a.ndim)

    return pl.pallas_call(
        _post_body,
        grid=grid,
        in_specs=[row_spec, row_spec, row_spec] + [full(a) for a in
                  (Wo, bo, W1, b1, W2, b2, g1, be1, g2, be2)],
        out_specs=row_spec,
        out_shape=jax.ShapeDtypeStruct((n, D), jnp.float32),
    )(out, recip, x, Wo, bo, W1, b1, W2, b2, g1, be1, g2, be2)


def kernel(x, edge_index, Wq, Wk, Wv, Wo, bo, W1, b1, W2, b2, g1, be1, g2, be2):
    n = x.shape[0]
    qh, kh, vh = _qkv_proj(x, Wq, Wk, Wv)
    qh = qh.reshape(2 * n, HALF)
    kh = kh.reshape(2 * n, HALF)
    vh = vh.reshape(2 * n, HALF)

    idt = edge_index.dtype
    loops = jnp.arange(n, dtype=idt)
    row_real = jnp.concatenate([edge_index[0], loops])
    col_real = jnp.concatenate([edge_index[1], loops])
    padz = jnp.zeros((_EP - _E_TOT,), dtype=idt)
    row_ext = jnp.concatenate([row_real, padz])
    col_ext = jnp.concatenate([col_real, padz])
    cols_raw = jnp.concatenate(
        [col_real, jnp.full((_EP - _E_TOT,), -1, dtype=idt)])
    rowg = jnp.concatenate([row_ext, row_ext + n])
    colg = jnp.concatenate([col_ext, col_ext + n])
    zeros_src = jnp.zeros((_RDL, HALF), jnp.float32)

    acc, sums_pk = _edge_kernel(rowg, colg, cols_raw, qh, kh, vh, zeros_src)

    out_nodes = jnp.concatenate([acc[:n], acc[n:]], axis=1)

    def sums_block(cc, tt):
        blk = sums_pk[(cc * 2 + tt) * _SROWS:(cc * 2 + tt + 1) * _SROWS]
        return blk.reshape(_SROWS * 64, 2)[:_NL]

    s01 = jnp.concatenate([sums_block(0, 0), sums_block(0, 1)], axis=0)
    s23 = jnp.concatenate([sums_block(1, 0), sums_block(1, 1)], axis=0)
    sums4 = jnp.concatenate([s01, s23], axis=1)
    recip = jnp.repeat(1.0 / (sums4 + 1e-8), HD, axis=1)

    return _post(out_nodes, recip, x, Wo, bo, W1, b1, W2, b2, g1, be1, g2, be2)


# speedup vs baseline: 1.4475x; 1.4475x over previous
import functools
import math

import jax
import jax.numpy as jnp
from jax import lax
from jax.experimental import pallas as pl
from jax.experimental.pallas import tpu as pltpu
from jax.experimental.pallas import tpu_sc as plsc

N_NODES = 10000
D = 256
H = 4
HD = D // H
HALF = D // 2

_BN = 1000

_C = 32
_NTILE = 16
_E_TOT = 160000 + N_NODES
_EPT = 10752
_EP = _EPT * _NTILE
_NCHUNK = _EPT // _C

_NL = 5000
_SINK = _NL
_RDL = 320
_SROWS = 80


def _qkv_body(x_ref, wq_ref, wk_ref, wv_ref, q_ref, k_ref, v_ref):
    x = x_ref[...]
    for w_ref, o_ref in ((wq_ref, q_ref), (wk_ref, k_ref), (wv_ref, v_ref)):
        full = jax.lax.dot_general(
            x, w_ref[...], (((1,), (1,)), ((), ())),
            preferred_element_type=jnp.float32)
        o_ref[0, ...] = full[:, :HALF]
        o_ref[1, ...] = full[:, HALF:]


def _qkv_proj(x, Wq, Wk, Wv):
    n = x.shape[0]
    grid = (n // _BN,)
    row_spec = pl.BlockSpec((_BN, D), lambda i: (i, 0))
    w_spec = pl.BlockSpec((D, D), lambda i: (0, 0))
    half_spec = pl.BlockSpec((2, _BN, HALF), lambda i: (0, i, 0))
    return pl.pallas_call(
        _qkv_body,
        grid=grid,
        in_specs=[row_spec, w_spec, w_spec, w_spec],
        out_specs=[half_spec] * 3,
        out_shape=[jax.ShapeDtypeStruct((2, n, HALF), jnp.float32)] * 3,
    )(x, Wq, Wk, Wv)


_sc_mesh = plsc.VectorSubcoreMesh(core_axis_name="c", subcore_axis_name="s")


@functools.partial(
    pl.kernel,
    mesh=_sc_mesh,
    out_type=[
        jax.ShapeDtypeStruct((2 * N_NODES, HALF), jnp.float32),
        jax.ShapeDtypeStruct((4 * _SROWS, HALF), jnp.float32),
    ],
    scratch_types=[
        pltpu.VMEM((2, _C), jnp.int32),
        pltpu.VMEM((2, _C), jnp.int32),
        pltpu.VMEM((2, _C), jnp.int32),
        pltpu.VMEM((2, _C), jnp.int32),
        pltpu.VMEM((2, _C), jnp.int32),
        pltpu.VMEM((2 * _C, HALF), jnp.float32),
        pltpu.VMEM((2 * _C, HALF), jnp.float32),
        pltpu.VMEM((2 * _C, HALF), jnp.float32),
        pltpu.VMEM((_C, HALF), jnp.float32),
        pltpu.VMEM((_C, HALF), jnp.float32),
        pltpu.VMEM_SHARED((_NL + 8, HALF), jnp.float32),
        pltpu.VMEM_SHARED((_SROWS, HALF), jnp.float32),
        pltpu.SemaphoreType.DMA,
        pltpu.SemaphoreType.DMA,
        pltpu.SemaphoreType.DMA,
    ],
)
def _edge_kernel(rowg_hbm, colg_hbm, cols_hbm, qh_hbm, kh_hbm, vh_hbm,
                 zeros_hbm, acc_out, sums_out,
                 rowg_v, colg_v, cols_v, colsS_v, srow_v, qbuf, kbuf, vbuf,
                 mbuf, msum, acc_sh, sums_sh, gsem0, gsem1, ssem):
    c = lax.axis_index("c")
    s = lax.axis_index("s")
    gsems = (gsem0, gsem1)

    lanes = lax.iota(jnp.int32, 16)
    base_idx = c * _EP + s * _EPT
    base_raw0 = s * _EPT

    def stage_and_fire(ch, b):
        base = base_idx + ch * _C
        base_raw = base_raw0 + ch * _C
        pltpu.sync_copy(rowg_hbm.at[pl.ds(base, _C)], rowg_v.at[b])
        pltpu.sync_copy(colg_hbm.at[pl.ds(base, _C)], colg_v.at[b])
        pltpu.sync_copy(cols_hbm.at[pl.ds(base_raw, _C)], cols_v.at[b])
        pltpu.async_copy(qh_hbm.at[rowg_v.at[b]],
                         qbuf.at[pl.ds(b * _C, _C)], gsems[b])
        pltpu.async_copy(kh_hbm.at[colg_v.at[b]],
                         kbuf.at[pl.ds(b * _C, _C)], gsems[b])
        pltpu.async_copy(vh_hbm.at[rowg_v.at[b]],
                         vbuf.at[pl.ds(b * _C, _C)], gsems[b])

    def wait_gathers(b):
        pltpu.make_async_copy(qh_hbm.at[rowg_v.at[b]],
                              qbuf.at[pl.ds(b * _C, _C)], gsems[b]).wait()
        pltpu.make_async_copy(kh_hbm.at[colg_v.at[b]],
                              kbuf.at[pl.ds(b * _C, _C)], gsems[b]).wait()
        pltpu.make_async_copy(vh_hbm.at[rowg_v.at[b]],
                              vbuf.at[pl.ds(b * _C, _C)], gsems[b]).wait()

    def wait_scatters(b):
        pltpu.make_async_copy(mbuf, acc_sh.at[colsS_v.at[b]], ssem).wait()
        pltpu.make_async_copy(msum, sums_sh.at[srow_v.at[b]], ssem).wait()

    def compute(t, b):
        roff = b * _C

        zero = jnp.zeros((16,), jnp.float32)

        def group_body(g, carry2):
            e0 = g * 16
            colraw = cols_v[b, pl.ds(e0, 16)]
            loc = colraw - t * _NL
            col16 = jnp.where((loc >= 0) & (loc < _NL), loc, _SINK)
            srow_v[b, pl.ds(e0, 16)] = col16 >> 6
            colsS_v[b, pl.ds(e0, 16)] = col16
            inrow16 = (col16 & 63) * 2

            for e in range(16):
                eg = e0 + e
                er = roff + eg
                acc0 = zero
                acc1 = zero
                for j in range(4):
                    acc0 = acc0 + (qbuf[er, pl.ds(j * 16, 16)]
                                   * kbuf[er, pl.ds(j * 16, 16)])
                    acc1 = acc1 + (qbuf[er, pl.ds(64 + j * 16, 16)]
                                   * kbuf[er, pl.ds(64 + j * 16, 16)])
                for kk in (1, 2, 4, 8):
                    perm = lanes ^ kk
                    acc0 = acc0 + acc0.at[perm].get(mode="promise_in_bounds")
                    acc1 = acc1 + acc1.at[perm].get(mode="promise_in_bounds")
                p0all = jnp.exp(acc0 * 0.125)
                p1all = jnp.exp(acc1 * 0.125)
                ev = jnp.zeros((16,), jnp.int32) + e
                inrow = inrow16.at[ev].get(mode="promise_in_bounds")
                inlane = inrow & 15
                jbv = inrow >> 4
                ohbase = (jnp.where(lanes == inlane, p0all, 0.0)
                          + jnp.where(lanes == inlane + 1, p1all, 0.0))
                for j in range(8):
                    pe = p0all if j < 4 else p1all
                    mbuf[eg, pl.ds(j * 16, 16)] = (
                        vbuf[er, pl.ds(j * 16, 16)] * pe)
                    dz = jnp.minimum(jnp.abs(jbv - j), 1)
                    msum[eg, pl.ds(j * 16, 16)] = ohbase * (
                        1.0 - dz.astype(jnp.float32))
            return carry2

        lax.fori_loop(0, _C // 16, group_body, 0)

    def phase_body(t, carry):
        @pl.when(s < _NTILE - 1)
        def _():
            pltpu.sync_copy(zeros_hbm.at[pl.ds(0, _RDL)],
                            acc_sh.at[pl.ds(s * _RDL, _RDL)])

        @pl.when(s == _NTILE - 1)
        def _():
            pltpu.sync_copy(zeros_hbm.at[pl.ds(0, 200)],
                            acc_sh.at[pl.ds(s * _RDL, 200)])

        @pl.when(s == 0)
        def _():
            pltpu.sync_copy(zeros_hbm.at[pl.ds(0, _SROWS)], sums_sh)

        plsc.subcore_barrier()

        stage_and_fire(0, 0)

        def pair_body(i, carry2):
            for b in range(2):
                ch = i * 2 + b
                nb = b ^ 1

                @pl.when(ch + 1 < _NCHUNK)
                def _():
                    stage_and_fire(ch + 1, nb)

                wait_gathers(b)

                @pl.when(ch > 0)
                def _():
                    wait_scatters(nb)

                compute(t, b)
                pltpu.async_copy(mbuf, acc_sh.at[colsS_v.at[b]], ssem,
                                 add=True)
                pltpu.async_copy(msum, sums_sh.at[srow_v.at[b]], ssem,
                                 add=True)
            return carry2

        lax.fori_loop(0, _NCHUNK // 2, pair_body, 0)
        wait_scatters(1)

        plsc.subcore_barrier()

        obase = c * N_NODES + t * _NL

        @pl.when(s < _NTILE - 1)
        def _():
            pltpu.sync_copy(acc_sh.at[pl.ds(s * _RDL, _RDL)],
                            acc_out.at[pl.ds(obase + s * _RDL, _RDL)])

        @pl.when(s == _NTILE - 1)
        def _():
            pltpu.sync_copy(acc_sh.at[pl.ds(s * _RDL, 200)],
                            acc_out.at[pl.ds(obase + s * _RDL, 200)])

        @pl.when(s == 0)
        def _():
            pltpu.sync_copy(
                sums_sh,
                sums_out.at[pl.ds((c * 2 + t) * _SROWS, _SROWS)])

        plsc.subcore_barrier()
        return carry

    lax.fori_loop(0, 2, phase_body, 0)


def _ln(x, g, b):
    mu = jnp.mean(x, axis=-1, keepdims=True)
    var = jnp.mean((x - mu) ** 2, axis=-1, keepdims=True)
    return (x - mu) * jax.lax.rsqrt(var + 1e-5) * g + b


def _erf(z):
    sg = jnp.sign(z)
    z = jnp.abs(z)
    t = 1.0 / (1.0 + 0.3275911 * z)
    poly = t * (0.254829592 + t * (-0.284496736 + t * (1.421413741
               + t * (-1.453152027 + t * 1.061405429))))
    return sg * (1.0 - poly * jnp.exp(-z * z))


def _post_body(out_ref, recip_ref, x_ref, wo_ref, bo_ref, w1_ref, b1_ref,
               w2_ref, b2_ref, g1_ref, be1_ref, g2_ref, be2_ref, y_ref):
    out = out_ref[...] * recip_ref[...]
    attn = jax.lax.dot_general(
        out, wo_ref[...], (((1,), (1,)), ((), ())),
        preferred_element_type=jnp.float32) + bo_ref[...]
    x1 = _ln(x_ref[...] + attn, g1_ref[...], be1_ref[...])
    h = jax.lax.dot_general(
        x1, w1_ref[...], (((1,), (1,)), ((), ())),
        preferred_element_type=jnp.float32) + b1_ref[...]
    h = h * 0.5 * (1.0 + _erf(h * (1.0 / math.sqrt(2.0))))
    ffn = jax.lax.dot_general(
        h, w2_ref[...], (((1,), (1,)), ((), ())),
        preferred_element_type=jnp.float32) + b2_ref[...]
    y_ref[...] = _ln(x1 + ffn, g2_ref[...], be2_ref[...])


def _post(out, recip, x, Wo, bo, W1, b1, W2, b2, g1, be1, g2, be2):
    n = x.shape[0]
    bo, b1, b2, g1, be1, g2, be2 = (a.reshape(1, -1) for a in
                                    (bo, b1, b2, g1, be1, g2, be2))
    grid = (n // _BN,)
    row_spec = pl.BlockSpec((_BN, D), lambda i: (i, 0))

    def full(a):
        return pl.BlockSpec(a.shape, lambda i: (0,) * a.ndim)

    return pl.pallas_call(
        _post_body,
        grid=grid,
        in_specs=[row_spec, row_spec, row_spec] + [full(a) for a in
                  (Wo, bo, W1, b1, W2, b2, g1, be1, g2, be2)],
        out_specs=row_spec,
        out_shape=jax.ShapeDtypeStruct((n, D), jnp.float32),
    )(out, recip, x, Wo, bo, W1, b1, W2, b2, g1, be1, g2, be2)


def kernel(x, edge_index, Wq, Wk, Wv, Wo, bo, W1, b1, W2, b2, g1, be1, g2, be2):
    n = x.shape[0]
    qh, kh, vh = _qkv_proj(x, Wq, Wk, Wv)
    qh = qh.reshape(2 * n, HALF)
    kh = kh.reshape(2 * n, HALF)
    vh = vh.reshape(2 * n, HALF)

    idt = edge_index.dtype
    loops = jnp.arange(n, dtype=idt)
    row_real = jnp.concatenate([edge_index[0], loops])
    col_real = jnp.concatenate([edge_index[1], loops])
    padz = jnp.zeros((_EP - _E_TOT,), dtype=idt)
    row_ext = jnp.concatenate([row_real, padz])
    col_ext = jnp.concatenate([col_real, padz])
    cols_raw = jnp.concatenate(
        [col_real, jnp.full((_EP - _E_TOT,), -1, dtype=idt)])
    rowg = jnp.concatenate([row_ext, row_ext + n])
    colg = jnp.concatenate([col_ext, col_ext + n])
    zeros_src = jnp.zeros((_RDL, HALF), jnp.float32)

    acc, sums_pk = _edge_kernel(rowg, colg, cols_raw, qh, kh, vh, zeros_src)

    out_nodes = jnp.concatenate([acc[:n], acc[n:]], axis=1)

    def sums_block(cc, tt):
        blk = sums_pk[(cc * 2 + tt) * _SROWS:(cc * 2 + tt + 1) * _SROWS]
        return blk.reshape(_SROWS * 64, 2)[:_NL]

    s01 = jnp.concatenate([sums_block(0, 0), sums_block(0, 1)], axis=0)
    s23 = jnp.concatenate([sums_block(1, 0), sums_block(1, 1)], axis=0)
    sums4 = jnp.concatenate([s01, s23], axis=1)
    recip = jnp.repeat(1.0 / (sums4 + 1e-8), HD, axis=1)

    return _post(out_nodes, recip, x, Wo, bo, W1, b1, W2, b2, g1, be1, g2, be2)
